# baseline (speedup 1.0000x reference)
import functools

import jax
import jax.numpy as jnp
from jax import lax
from jax.experimental import pallas as pl
from jax.experimental.pallas import tpu as pltpu
from jax.experimental.pallas import tpu_sc as plsc

B = 16384
F = 100
D = 32
NCAT = 10000
TOTAL = B * F

NC, NS, L = 2, 16, 16
NW = NC * NS
SPAN = TOTAL // NW

CHUNK = 1280
NCHUNK = SPAN // CHUNK


def _tokenizer_gather(xf, table):
    mesh = plsc.VectorSubcoreMesh(core_axis_name="c", subcore_axis_name="s")

    @functools.partial(
        pl.kernel,
        out_type=jax.ShapeDtypeStruct((TOTAL, D), jnp.float32),
        mesh=mesh,
        scratch_types=[
            pltpu.VMEM((2, CHUNK), jnp.int32),
            pltpu.VMEM((2, CHUNK), jnp.int32),
            pltpu.VMEM((2, CHUNK, D), jnp.float32),
            pltpu.SemaphoreType.DMA,
            pltpu.SemaphoreType.DMA,
            pltpu.SemaphoreType.DMA,
            pltpu.SemaphoreType.DMA,
            pltpu.SemaphoreType.DMA,
            pltpu.SemaphoreType.DMA,
        ],
        compiler_params=pltpu.CompilerParams(use_tc_tiling_on_sc=False),
    )
    def k(x_hbm, table_hbm, out_hbm, xv, idxv, rows_v,
          si0, si1, sg0, sg1, so0, so1):
        wid = lax.axis_index("s") * NC + lax.axis_index("c")
        flat_base = wid * SPAN
        iota = lax.iota(jnp.int32, L)
        si = (si0, si1)
        sg = (sg0, sg1)
        so = (so0, so1)

        def in_cp(g, b):
            return pltpu.make_async_copy(
                x_hbm.at[pl.ds(flat_base + g * CHUNK, CHUNK)],
                xv.at[b], si[b])

        def gat_cp(b):
            return pltpu.make_async_copy(
                table_hbm.at[idxv.at[b]], rows_v.at[b], sg[b])

        def out_cp(g, b):
            return pltpu.make_async_copy(
                rows_v.at[b],
                out_hbm.at[pl.ds(flat_base + g * CHUNK, CHUNK)], so[b])

        def body(g, b):
            nb = 1 - b
            in_cp(g, b).wait()

            @pl.when(g + 1 < NCHUNK)
            def _():
                in_cp(g + 1, nb).start()

            flat0 = flat_base + g * CHUNK
            for kk in range(CHUNK // L):
                pos = iota + (flat0 + kk * L)
                off = lax.rem(pos, F) * NCAT
                sl = pl.ds(kk * L, L)
                idxv[b, sl] = xv[b, sl] + off

            @pl.when(g >= 1)
            def _():
                gat_cp(nb).wait()
                out_cp(g - 1, nb).start()

            @pl.when(g >= 2)
            def _():
                out_cp(g - 2, b).wait()

            gat_cp(b).start()

        in_cp(0, 0).start()

        def pair(i, _):
            body(2 * i, 0)
            body(2 * i + 1, 1)
            return 0

        lax.fori_loop(0, NCHUNK // 2, pair, 0)

        last_b = (NCHUNK - 1) % 2
        gat_cp(last_b).wait()
        out_cp(NCHUNK - 1, last_b).start()
        out_cp(NCHUNK - 2, 1 - last_b).wait()
        out_cp(NCHUNK - 1, last_b).wait()

    return k(xf, table)


@jax.jit
def kernel(x, table, bias):
    del bias
    xf = x.reshape(TOTAL)
    out = _tokenizer_gather(xf, table)
    return out.reshape(B, F, D)


# speedup vs baseline: 1.0002x; 1.0002x over previous
import functools

import jax
import jax.numpy as jnp
from jax import lax
from jax.experimental import pallas as pl
from jax.experimental.pallas import tpu as pltpu
from jax.experimental.pallas import tpu_sc as plsc

B = 16384
F = 100
D = 32
NCAT = 10000
TOTAL = B * F

NC, NS, L = 2, 16, 16
NW = NC * NS
SPAN = TOTAL // NW

IDX_W = 128
CHUNK_ROWS = 10
CHUNK = CHUNK_ROWS * IDX_W
NCHUNK = SPAN // CHUNK


def _tokenizer_gather(xf, table):
    mesh = plsc.VectorSubcoreMesh(core_axis_name="c", subcore_axis_name="s")

    @functools.partial(
        pl.kernel,
        out_type=jax.ShapeDtypeStruct((TOTAL, D), jnp.float32),
        mesh=mesh,
        scratch_types=[
            pltpu.VMEM((2, CHUNK), jnp.int32),
            pltpu.VMEM((2, CHUNK), jnp.int32),
            pltpu.VMEM((2, CHUNK, D), jnp.float32),
            pltpu.SemaphoreType.DMA,
            pltpu.SemaphoreType.DMA,
            pltpu.SemaphoreType.DMA,
            pltpu.SemaphoreType.DMA,
            pltpu.SemaphoreType.DMA,
            pltpu.SemaphoreType.DMA,
        ],
        compiler_params=pltpu.CompilerParams(use_tc_tiling_on_sc=False),
    )
    def k(x_hbm, table_hbm, out_hbm, xv, idxv, rows_v,
          si0, si1, sg0, sg1, so0, so1):
        wid = lax.axis_index("s") * NC + lax.axis_index("c")
        flat_base = wid * SPAN
        iota = lax.iota(jnp.int32, L)
        si = (si0, si1)
        sg = (sg0, sg1)
        so = (so0, so1)

        def in_cp(g, b):
            return pltpu.make_async_copy(
                x_hbm.at[pl.ds(flat_base + g * CHUNK, CHUNK)],
                xv.at[b], si[b])

        def gat_cps(b):
            return [
                pltpu.make_async_copy(
                    table_hbm.at[idxv.at[b, pl.ds(r * IDX_W, IDX_W)]],
                    rows_v.at[b, pl.ds(r * IDX_W, IDX_W)], sg[b])
                for r in range(CHUNK_ROWS)
            ]

        def out_cp(g, b):
            return pltpu.make_async_copy(
                rows_v.at[b],
                out_hbm.at[pl.ds(flat_base + g * CHUNK, CHUNK)], so[b])

        def body(g, b):
            nb = 1 - b
            in_cp(g, b).wait()

            @pl.when(g + 1 < NCHUNK)
            def _():
                in_cp(g + 1, nb).start()

            flat0 = flat_base + g * CHUNK
            for kk in range(CHUNK // L):
                pos = iota + (flat0 + kk * L)
                off = lax.rem(pos, F) * NCAT
                sl = pl.ds(kk * L, L)
                idxv[b, sl] = xv[b, sl] + off

            @pl.when(g >= 1)
            def _():
                for c in gat_cps(nb):
                    c.wait()
                out_cp(g - 1, nb).start()

            @pl.when(g >= 2)
            def _():
                out_cp(g - 2, b).wait()

            for c in gat_cps(b):
                c.start()

        in_cp(0, 0).start()

        def pair(i, _):
            body(2 * i, 0)
            body(2 * i + 1, 1)
            return 0

        lax.fori_loop(0, NCHUNK // 2, pair, 0)

        last_b = (NCHUNK - 1) % 2
        for c in gat_cps(last_b):
            c.wait()
        out_cp(NCHUNK - 1, last_b).start()
        out_cp(NCHUNK - 2, 1 - last_b).wait()
        out_cp(NCHUNK - 1, last_b).wait()

    return k(xf, table)


@jax.jit
def kernel(x, table, bias):
    del bias
    xf = x.reshape(TOTAL)
    out = _tokenizer_gather(xf, table)
    return out.reshape(B, F, D)
